# baseline (speedup 1.0000x reference)
import functools

import jax
import jax.numpy as jnp
from jax import lax
from jax.experimental import pallas as pl
from jax.experimental.pallas import tpu as pltpu
from jax.experimental.pallas import tpu_sc as plsc

_DIM = 2048
_LORA_DIMS = (8, 16, 32, 48, 64, 96, 128)
_NE = 7
_STACK = 512
_STARTS = (0, 8, 24, 56, 104, 168, 264)
_BOUNDS = (8, 24, 56, 104, 168, 264, 392)
_BLK = 1024
_TOKENS = 8192
_BLK2 = 256
_P = _TOKENS + _NE * _BLK2
_NW = 32


def _stage1_body(x_ref, p_ref, *rest):
    wa_refs = rest[0:_NE]
    h_ref = rest[_NE]
    e_ref = rest[_NE + 1]
    a_s = rest[_NE + 2]

    @pl.when(pl.program_id(0) == 0)
    def _assemble():
        a_s[...] = jnp.zeros_like(a_s)
        for i in range(_NE):
            s, d = _STARTS[i], _LORA_DIMS[i]
            a_s[s:s + d, :] = wa_refs[i][...]

    x = x_ref[...]
    logits = jax.lax.dot_general(
        x.astype(jnp.bfloat16), p_ref[...].astype(jnp.bfloat16),
        (((1,), (1,)), ((), ())),
        preferred_element_type=jnp.float32,
        precision=jax.lax.Precision.DEFAULT)
    col8 = jax.lax.broadcasted_iota(jnp.int32, logits.shape, 1)
    logits = jnp.where(col8 < _NE, logits, jnp.float32(-3e38))
    m = jnp.max(logits, axis=1, keepdims=True)
    e = jnp.min(jnp.where(logits >= m, col8, _NE), axis=1, keepdims=True)

    h = jax.lax.dot_general(
        x, a_s[...], (((1,), (1,)), ((), ())),
        preferred_element_type=jnp.float32, precision=jax.lax.Precision.DEFAULT)
    col128 = jax.lax.broadcasted_iota(jnp.int32, (x.shape[0], 128), 1)
    hsel = jnp.zeros((x.shape[0], 128), jnp.float32)
    for i in range(_NE):
        s, d = _STARTS[i], _LORA_DIMS[i]
        piece = h[:, s:s + 128]
        piece = jnp.where((col128 < d) & (e == i), piece, jnp.float32(0.0))
        hsel = hsel + piece
    h_ref[...] = hsel
    e_ref[...] = jnp.broadcast_to(e, (x.shape[0], 128))


def _stage1(x, p_pad, was):
    n_blk = x.shape[0] // _BLK
    const_spec = lambda shape: pl.BlockSpec(shape, lambda i: (0,) * len(shape))
    return pl.pallas_call(
        _stage1_body,
        grid=(n_blk,),
        in_specs=[
            pl.BlockSpec((_BLK, _DIM), lambda i: (i, 0)),
            const_spec((8, _DIM)),
        ] + [const_spec((d, _DIM)) for d in _LORA_DIMS],
        out_specs=[
            pl.BlockSpec((_BLK, 128), lambda i: (i, 0)),
            pl.BlockSpec((_BLK, 128), lambda i: (i, 0)),
        ],
        out_shape=[
            jax.ShapeDtypeStruct((_TOKENS, 128), jnp.float32),
            jax.ShapeDtypeStruct((_TOKENS, 128), jnp.int32),
        ],
        scratch_shapes=[pltpu.VMEM((_STACK, _DIM), jnp.float32)],
        compiler_params=pltpu.CompilerParams(
            dimension_semantics=("arbitrary",)),
    )(x, p_pad, *was)


def _sc_scatter_rows(rows, dst):
    n_per_w = _TOKENS // _NW
    mesh = plsc.VectorSubcoreMesh(core_axis_name="c", subcore_axis_name="s")

    @functools.partial(
        pl.kernel, mesh=mesh,
        out_type=jax.ShapeDtypeStruct((_P, 128), jnp.float32),
        scratch_types=[
            pltpu.VMEM((n_per_w,), jnp.int32),
            pltpu.VMEM((n_per_w, 128), jnp.float32),
            pltpu.SemaphoreType.DMA,
        ],
    )
    def k(rows_hbm, dst_hbm, out_hbm, idx_v, rows_v, sem):
        wid = lax.axis_index("s") * 2 + lax.axis_index("c")
        base = wid * n_per_w
        pltpu.sync_copy(dst_hbm.at[pl.ds(base, n_per_w)], idx_v)
        pltpu.sync_copy(rows_hbm.at[pl.ds(base, n_per_w)], rows_v)
        for c in range(0, n_per_w, 128):
            pltpu.async_copy(
                rows_v.at[pl.ds(c, 128)],
                out_hbm.at[idx_v.at[pl.ds(c, 128)]], sem).wait()

    return k(rows, dst)


def _sc_gather_rows(table, dst):
    n_per_w = _TOKENS // _NW
    chunk = 32
    mesh = plsc.VectorSubcoreMesh(core_axis_name="c", subcore_axis_name="s")

    @functools.partial(
        pl.kernel, mesh=mesh,
        out_type=jax.ShapeDtypeStruct((_TOKENS, _DIM), jnp.float32),
        scratch_types=[
            pltpu.VMEM((n_per_w,), jnp.int32),
            pltpu.VMEM((chunk, _DIM), jnp.float32),
            pltpu.SemaphoreType.DMA,
        ],
    )
    def k(table_hbm, dst_hbm, out_hbm, idx_v, rows_v, sem):
        wid = lax.axis_index("s") * 2 + lax.axis_index("c")
        base = wid * n_per_w
        pltpu.sync_copy(dst_hbm.at[pl.ds(base, n_per_w)], idx_v)
        for c in range(0, n_per_w, chunk):
            pltpu.async_copy(
                table_hbm.at[idx_v.at[pl.ds(c, chunk)]], rows_v, sem).wait()
            pltpu.sync_copy(rows_v, out_hbm.at[pl.ds(base + c, chunk)])

    return k(table, dst)


def _stage4_body(be_ref, hs_ref, w_ref, o_ref):
    o_ref[...] = jax.lax.dot_general(
        hs_ref[...], w_ref[0], (((1,), (0,)), ((), ())),
        preferred_element_type=jnp.float32, precision=jax.lax.Precision.DEFAULT)


def _stage4(hs, wbt, block_expert):
    n_blk = _P // _BLK2
    grid_spec = pltpu.PrefetchScalarGridSpec(
        num_scalar_prefetch=1,
        grid=(n_blk,),
        in_specs=[
            pl.BlockSpec((_BLK2, 128), lambda j, be: (j, 0)),
            pl.BlockSpec((1, 128, _DIM), lambda j, be: (be[j], 0, 0)),
        ],
        out_specs=pl.BlockSpec((_BLK2, _DIM), lambda j, be: (j, 0)),
    )
    return pl.pallas_call(
        _stage4_body,
        grid_spec=grid_spec,
        out_shape=jax.ShapeDtypeStruct((_P, _DIM), jnp.float32),
        compiler_params=pltpu.CompilerParams(
            dimension_semantics=("arbitrary",)),
    )(block_expert, hs, wbt)


def kernel(x, prototypes, Wa0, Wa1, Wa2, Wa3, Wa4, Wa5, Wa6,
           Wb0, Wb1, Wb2, Wb3, Wb4, Wb5, Wb6):
    was = [Wa0, Wa1, Wa2, Wa3, Wa4, Wa5, Wa6]
    wbs = [Wb0, Wb1, Wb2, Wb3, Wb4, Wb5, Wb6]
    p_pad = jnp.pad(prototypes, ((0, 8 - _NE), (0, 0)))

    hsel, e128 = _stage1(x, p_pad, was)
    e = e128[:, 0]

    onehot = (e[:, None] == jnp.arange(_NE)[None, :]).astype(jnp.int32)
    counts = jnp.sum(onehot, axis=0)
    rank_ex = jnp.cumsum(onehot, axis=0) - onehot
    rank = jnp.sum(rank_ex * onehot, axis=1)
    cap = ((counts + _BLK2 - 1) // _BLK2) * _BLK2
    pend = jnp.cumsum(cap)
    pstart = pend - cap
    dst = (pstart[e] + rank).astype(jnp.int32)

    blk_start = jnp.arange(_P // _BLK2, dtype=jnp.int32) * _BLK2
    block_expert = jnp.minimum(jnp.sum(
        (blk_start[:, None] >= pend[None, :]).astype(jnp.int32), axis=1),
        _NE - 1)

    hs = _sc_scatter_rows(hsel, dst)

    wbt = jnp.stack([
        jnp.pad(w.T, ((0, 128 - w.shape[1]), (0, 0))) for w in wbs])
    out_s = _stage4(hs, wbt, block_expert)

    return _sc_gather_rows(out_s, dst)


# speedup vs baseline: 2.1480x; 2.1480x over previous
import jax
import jax.numpy as jnp
from jax.experimental import pallas as pl
from jax.experimental.pallas import tpu as pltpu

_DIM = 2048
_LORA_DIMS = (8, 16, 32, 48, 64, 96, 128)
_NE = 7
_STACK = 512
_STARTS = (0, 8, 24, 56, 104, 168, 264)
_BOUNDS = (8, 24, 56, 104, 168, 264, 392)
_BLK = 1024

_EXP_PREC = jax.lax.Precision.DEFAULT


def _moe_body(x_ref, p_ref, *rest):
    wa_refs = rest[0:_NE]
    wb_refs = rest[_NE:2 * _NE]
    o_ref = rest[2 * _NE]
    a_s = rest[2 * _NE + 1]
    b_s = rest[2 * _NE + 2]
    seg_s = rest[2 * _NE + 3]

    @pl.when(pl.program_id(0) == 0)
    def _assemble():
        a_s[...] = jnp.zeros_like(a_s)
        b_s[...] = jnp.zeros_like(b_s)
        for i in range(_NE):
            s, d = _STARTS[i], _LORA_DIMS[i]
            a_s[s:s + d, :] = wa_refs[i][...]
            b_s[:, s:s + d] = wb_refs[i][...]
        segc = jax.lax.broadcasted_iota(jnp.int32, (8, _STACK), 1)
        seg = jnp.zeros((8, _STACK), jnp.int32)
        for b in _BOUNDS:
            seg += (segc >= b).astype(jnp.int32)
        seg_s[...] = seg

    x = x_ref[...]
    logits = jax.lax.dot_general(
        x, p_ref[...], (((1,), (1,)), ((), ())),
        preferred_element_type=jnp.float32,
        precision=jax.lax.Precision.DEFAULT)
    col8 = jax.lax.broadcasted_iota(jnp.int32, logits.shape, 1)
    logits = jnp.where(col8 < _NE, logits, jnp.float32(-3e38))
    m = jnp.max(logits, axis=1, keepdims=True)
    e = jnp.min(jnp.where(logits >= m, col8, _NE), axis=1, keepdims=True)

    h = jax.lax.dot_general(
        x, a_s[...], (((1,), (1,)), ((), ())),
        preferred_element_type=jnp.float32, precision=_EXP_PREC)
    hm = jnp.where(seg_s[0:1, :] == e, h, jnp.float32(0.0))

    o_ref[...] = jax.lax.dot_general(
        hm, b_s[...], (((1,), (1,)), ((), ())),
        preferred_element_type=jnp.float32, precision=_EXP_PREC)


def kernel(x, prototypes, Wa0, Wa1, Wa2, Wa3, Wa4, Wa5, Wa6,
           Wb0, Wb1, Wb2, Wb3, Wb4, Wb5, Wb6):
    was = [Wa0, Wa1, Wa2, Wa3, Wa4, Wa5, Wa6]
    wbs = [Wb0, Wb1, Wb2, Wb3, Wb4, Wb5, Wb6]
    p_pad = jnp.pad(prototypes, ((0, 8 - _NE), (0, 0)))

    n_blk = x.shape[0] // _BLK
    const_spec = lambda shape: pl.BlockSpec(shape, lambda i: (0,) * len(shape))
    return pl.pallas_call(
        _moe_body,
        grid=(n_blk,),
        in_specs=[
            pl.BlockSpec((_BLK, _DIM), lambda i: (i, 0)),
            const_spec((8, _DIM)),
        ] + [const_spec((d, _DIM)) for d in _LORA_DIMS]
          + [const_spec((_DIM, d)) for d in _LORA_DIMS],
        out_specs=pl.BlockSpec((_BLK, _DIM), lambda i: (i, 0)),
        out_shape=jax.ShapeDtypeStruct((x.shape[0], _DIM), jnp.float32),
        scratch_shapes=[
            pltpu.VMEM((_STACK, _DIM), jnp.float32),
            pltpu.VMEM((_DIM, _STACK), jnp.float32),
            pltpu.VMEM((8, _STACK), jnp.int32),
        ],
        compiler_params=pltpu.CompilerParams(
            dimension_semantics=("arbitrary",)),
    )(x, p_pad, *was, *wbs)


# speedup vs baseline: 2.1533x; 1.0024x over previous
import jax
import jax.numpy as jnp
from jax.experimental import pallas as pl
from jax.experimental.pallas import tpu as pltpu

_DIM = 2048
_LORA_DIMS = (8, 16, 32, 48, 64, 96, 128)
_NE = 7
_STACK = 512
_STARTS = (0, 8, 24, 56, 104, 168, 264)
_BOUNDS = (8, 24, 56, 104, 168, 264, 392)
_BLK = 1024

_EXP_PREC = jax.lax.Precision.DEFAULT


def _moe_body(x_ref, p_ref, *rest):
    wa_refs = rest[0:_NE]
    wb_refs = rest[_NE:2 * _NE]
    o_ref = rest[2 * _NE]
    a_s = rest[2 * _NE + 1]
    b_s = rest[2 * _NE + 2]
    seg_s = rest[2 * _NE + 3]
    p_s = rest[2 * _NE + 4]

    @pl.when(pl.program_id(0) == 0)
    def _assemble():
        a_s[...] = jnp.zeros_like(a_s)
        b_s[...] = jnp.zeros_like(b_s)
        for i in range(_NE):
            s, d = _STARTS[i], _LORA_DIMS[i]
            a_s[s:s + d, :] = wa_refs[i][...].astype(jnp.bfloat16)
            b_s[:, s:s + d] = wb_refs[i][...].astype(jnp.bfloat16)
        p_s[...] = p_ref[...].astype(jnp.bfloat16)
        segc = jax.lax.broadcasted_iota(jnp.int32, (8, _STACK), 1)
        seg = jnp.zeros((8, _STACK), jnp.int32)
        for b in _BOUNDS:
            seg += (segc >= b).astype(jnp.int32)
        seg_s[...] = seg

    xb = x_ref[...].astype(jnp.bfloat16)
    logits = jax.lax.dot_general(
        xb, p_s[...], (((1,), (1,)), ((), ())),
        preferred_element_type=jnp.float32,
        precision=jax.lax.Precision.DEFAULT)
    col8 = jax.lax.broadcasted_iota(jnp.int32, logits.shape, 1)
    logits = jnp.where(col8 < _NE, logits, jnp.float32(-3e38))
    m = jnp.max(logits, axis=1, keepdims=True)
    e = jnp.min(jnp.where(logits >= m, col8, _NE), axis=1, keepdims=True)

    h = jax.lax.dot_general(
        xb, a_s[...], (((1,), (1,)), ((), ())),
        preferred_element_type=jnp.float32, precision=_EXP_PREC)
    hm = jnp.where(seg_s[0:1, :] == e, h.astype(jnp.bfloat16),
                   jnp.bfloat16(0.0))

    o_ref[...] = jax.lax.dot_general(
        hm, b_s[...], (((1,), (1,)), ((), ())),
        preferred_element_type=jnp.float32, precision=_EXP_PREC)


def kernel(x, prototypes, Wa0, Wa1, Wa2, Wa3, Wa4, Wa5, Wa6,
           Wb0, Wb1, Wb2, Wb3, Wb4, Wb5, Wb6):
    was = [Wa0, Wa1, Wa2, Wa3, Wa4, Wa5, Wa6]
    wbs = [Wb0, Wb1, Wb2, Wb3, Wb4, Wb5, Wb6]
    p_pad = jnp.pad(prototypes, ((0, 8 - _NE), (0, 0)))

    n_blk = x.shape[0] // _BLK
    const_spec = lambda shape: pl.BlockSpec(shape, lambda i: (0,) * len(shape))
    return pl.pallas_call(
        _moe_body,
        grid=(n_blk,),
        in_specs=[
            pl.BlockSpec((_BLK, _DIM), lambda i: (i, 0)),
            const_spec((8, _DIM)),
        ] + [const_spec((d, _DIM)) for d in _LORA_DIMS]
          + [const_spec((_DIM, d)) for d in _LORA_DIMS],
        out_specs=pl.BlockSpec((_BLK, _DIM), lambda i: (i, 0)),
        out_shape=jax.ShapeDtypeStruct((x.shape[0], _DIM), jnp.float32),
        scratch_shapes=[
            pltpu.VMEM((_STACK, _DIM), jnp.bfloat16),
            pltpu.VMEM((_DIM, _STACK), jnp.bfloat16),
            pltpu.VMEM((8, _STACK), jnp.int32),
            pltpu.VMEM((8, _DIM), jnp.bfloat16),
        ],
        compiler_params=pltpu.CompilerParams(
            dimension_semantics=("arbitrary",)),
    )(x, p_pad, *was, *wbs)


# speedup vs baseline: 2.3002x; 1.0682x over previous
import jax
import jax.numpy as jnp
from jax.experimental import pallas as pl
from jax.experimental.pallas import tpu as pltpu

_DIM = 2048
_LORA_DIMS = (8, 16, 32, 48, 64, 96, 128)
_NE = 7
_STACK = 512
_STARTS = (0, 8, 24, 56, 104, 168, 264)
_BOUNDS = (8, 24, 56, 104, 168, 264, 392)
_BLK = 1024

_EXP_PREC = jax.lax.Precision.DEFAULT


def _moe_body(x_ref, p_ref, *rest):
    wa_refs = rest[0:_NE]
    wb_refs = rest[_NE:2 * _NE]
    o_ref = rest[2 * _NE]
    a_s = rest[2 * _NE + 1]
    b_s = rest[2 * _NE + 2]
    seg_s = rest[2 * _NE + 3]
    p_s = rest[2 * _NE + 4]

    @pl.when(pl.program_id(0) == 0)
    def _assemble():
        a_s[...] = jnp.zeros_like(a_s)
        b_s[...] = jnp.zeros_like(b_s)
        for i in range(_NE):
            s, d = _STARTS[i], _LORA_DIMS[i]
            a_s[s:s + d, :] = wa_refs[i][...].astype(jnp.bfloat16)
            b_s[:, s:s + d] = wb_refs[i][...].astype(jnp.bfloat16)
        p_s[...] = p_ref[...].astype(jnp.bfloat16)
        segc = jax.lax.broadcasted_iota(jnp.int32, (8, _STACK), 1)
        seg = jnp.zeros((8, _STACK), jnp.int32)
        for b in _BOUNDS:
            seg += (segc >= b).astype(jnp.int32)
        seg_s[...] = seg

    half = _BLK // 2
    for sub in range(2):
        xb = x_ref[sub * half:(sub + 1) * half, :].astype(jnp.bfloat16)
        logits = jax.lax.dot_general(
            xb, p_s[...], (((1,), (1,)), ((), ())),
            preferred_element_type=jnp.float32,
            precision=jax.lax.Precision.DEFAULT)
        col8 = jax.lax.broadcasted_iota(jnp.int32, logits.shape, 1)
        logits = jnp.where(col8 < _NE, logits, jnp.float32(-3e38))
        m = jnp.max(logits, axis=1, keepdims=True)
        e = jnp.min(jnp.where(logits >= m, col8, _NE), axis=1, keepdims=True)

        h = jax.lax.dot_general(
            xb, a_s[...], (((1,), (1,)), ((), ())),
            preferred_element_type=jnp.float32, precision=_EXP_PREC)
        hm = jnp.where(seg_s[0:1, :] == e, h.astype(jnp.bfloat16),
                       jnp.bfloat16(0.0))

        o_ref[sub * half:(sub + 1) * half, :] = jax.lax.dot_general(
            hm, b_s[...], (((1,), (1,)), ((), ())),
            preferred_element_type=jnp.float32, precision=_EXP_PREC)


def kernel(x, prototypes, Wa0, Wa1, Wa2, Wa3, Wa4, Wa5, Wa6,
           Wb0, Wb1, Wb2, Wb3, Wb4, Wb5, Wb6):
    was = [Wa0, Wa1, Wa2, Wa3, Wa4, Wa5, Wa6]
    wbs = [Wb0, Wb1, Wb2, Wb3, Wb4, Wb5, Wb6]
    p_pad = jnp.pad(prototypes, ((0, 8 - _NE), (0, 0)))

    n_blk = x.shape[0] // _BLK
    const_spec = lambda shape: pl.BlockSpec(shape, lambda i: (0,) * len(shape))
    return pl.pallas_call(
        _moe_body,
        grid=(n_blk,),
        in_specs=[
            pl.BlockSpec((_BLK, _DIM), lambda i: (i, 0)),
            const_spec((8, _DIM)),
        ] + [const_spec((d, _DIM)) for d in _LORA_DIMS]
          + [const_spec((_DIM, d)) for d in _LORA_DIMS],
        out_specs=pl.BlockSpec((_BLK, _DIM), lambda i: (i, 0)),
        out_shape=jax.ShapeDtypeStruct((x.shape[0], _DIM), jnp.float32),
        scratch_shapes=[
            pltpu.VMEM((_STACK, _DIM), jnp.bfloat16),
            pltpu.VMEM((_DIM, _STACK), jnp.bfloat16),
            pltpu.VMEM((8, _STACK), jnp.int32),
            pltpu.VMEM((8, _DIM), jnp.bfloat16),
        ],
        compiler_params=pltpu.CompilerParams(
            dimension_semantics=("arbitrary",)),
    )(x, p_pad, *was, *wbs)


# speedup vs baseline: 2.3344x; 1.0148x over previous
import jax
import jax.numpy as jnp
from jax.experimental import pallas as pl
from jax.experimental.pallas import tpu as pltpu

_DIM = 2048
_LORA_DIMS = (8, 16, 32, 48, 64, 96, 128)
_NE = 7
_STACK = 512
_STARTS = (0, 8, 24, 56, 104, 168, 264)
_BOUNDS = (8, 24, 56, 104, 168, 264, 392)
_BLK = 1024

_EXP_PREC = jax.lax.Precision.DEFAULT


def _moe_body(x_ref, p_ref, *rest):
    wa_refs = rest[0:_NE]
    wb_refs = rest[_NE:2 * _NE]
    o_ref = rest[2 * _NE]
    a_s = rest[2 * _NE + 1]
    b_s = rest[2 * _NE + 2]
    seg_s = rest[2 * _NE + 3]
    p_s = rest[2 * _NE + 4]

    @pl.when(pl.program_id(0) == 0)
    def _assemble():
        a_s[...] = jnp.zeros_like(a_s)
        b_s[...] = jnp.zeros_like(b_s)
        for i in range(_NE):
            s, d = _STARTS[i], _LORA_DIMS[i]
            a_s[s:s + d, :] = wa_refs[i][...].astype(jnp.bfloat16)
            b_s[:, s:s + d] = wb_refs[i][...].astype(jnp.bfloat16)
        p_s[...] = p_ref[...].astype(jnp.bfloat16)
        segc = jax.lax.broadcasted_iota(jnp.int32, (8, _STACK), 1)
        seg = jnp.zeros((8, _STACK), jnp.int32)
        for b in _BOUNDS:
            seg += (segc >= b).astype(jnp.int32)
        seg_s[...] = seg

    half = _BLK // 4
    for sub in range(4):
        xb = x_ref[sub * half:(sub + 1) * half, :].astype(jnp.bfloat16)
        logits = jax.lax.dot_general(
            xb, p_s[...], (((1,), (1,)), ((), ())),
            preferred_element_type=jnp.float32,
            precision=jax.lax.Precision.DEFAULT)
        col8 = jax.lax.broadcasted_iota(jnp.int32, logits.shape, 1)
        logits = jnp.where(col8 < _NE, logits, jnp.float32(-3e38))
        m = jnp.max(logits, axis=1, keepdims=True)
        e = jnp.min(jnp.where(logits >= m, col8, _NE), axis=1, keepdims=True)

        h = jax.lax.dot_general(
            xb, a_s[...], (((1,), (1,)), ((), ())),
            preferred_element_type=jnp.float32, precision=_EXP_PREC)
        hm = jnp.where(seg_s[0:1, :] == e, h.astype(jnp.bfloat16),
                       jnp.bfloat16(0.0))

        o_ref[sub * half:(sub + 1) * half, :] = jax.lax.dot_general(
            hm, b_s[...], (((1,), (1,)), ((), ())),
            preferred_element_type=jnp.float32, precision=_EXP_PREC)


def kernel(x, prototypes, Wa0, Wa1, Wa2, Wa3, Wa4, Wa5, Wa6,
           Wb0, Wb1, Wb2, Wb3, Wb4, Wb5, Wb6):
    was = [Wa0, Wa1, Wa2, Wa3, Wa4, Wa5, Wa6]
    wbs = [Wb0, Wb1, Wb2, Wb3, Wb4, Wb5, Wb6]
    p_pad = jnp.pad(prototypes, ((0, 8 - _NE), (0, 0)))

    n_blk = x.shape[0] // _BLK
    const_spec = lambda shape: pl.BlockSpec(shape, lambda i: (0,) * len(shape))
    return pl.pallas_call(
        _moe_body,
        grid=(n_blk,),
        in_specs=[
            pl.BlockSpec((_BLK, _DIM), lambda i: (i, 0)),
            const_spec((8, _DIM)),
        ] + [const_spec((d, _DIM)) for d in _LORA_DIMS]
          + [const_spec((_DIM, d)) for d in _LORA_DIMS],
        out_specs=pl.BlockSpec((_BLK, _DIM), lambda i: (i, 0)),
        out_shape=jax.ShapeDtypeStruct((x.shape[0], _DIM), jnp.float32),
        scratch_shapes=[
            pltpu.VMEM((_STACK, _DIM), jnp.bfloat16),
            pltpu.VMEM((_DIM, _STACK), jnp.bfloat16),
            pltpu.VMEM((8, _STACK), jnp.int32),
            pltpu.VMEM((8, _DIM), jnp.bfloat16),
        ],
        compiler_params=pltpu.CompilerParams(
            dimension_semantics=("arbitrary",)),
    )(x, p_pad, *was, *wbs)
